# baseline (speedup 1.0000x reference)
import functools

import jax
import jax.numpy as jnp
from jax import lax
from jax.experimental import pallas as pl
from jax.experimental.pallas import tpu as pltpu
from jax.experimental.pallas import tpu_sc as plsc

B = 16384
D = 64
NC = 2
NS = 16
NW = NC * NS
BPW = B // NW
NCHUNK = 4
CH = BPW // NCHUNK


def _sc_body(src_hbm, tgt_hbm, table_hbm, wb_hbm, out_hbm,
             sidx, tidx, srows, trows, outv, wbv, sem):
    cid = lax.axis_index("c")
    sid = lax.axis_index("s")
    wid = sid * NC + cid
    base = wid * BPW

    pltpu.sync_copy(src_hbm.at[pl.ds(base, BPW)], sidx)
    pltpu.sync_copy(tgt_hbm.at[pl.ds(base, BPW)], tidx)
    pltpu.sync_copy(wb_hbm, wbv)

    copies = []
    for j in range(NCHUNK):
        copies.append(pltpu.async_copy(
            table_hbm.at[sidx.at[pl.ds(j * CH, CH)]],
            srows.at[pl.ds(j * CH, CH)], sem))
        copies.append(pltpu.async_copy(
            table_hbm.at[tidx.at[pl.ds(j * CH, CH)]],
            trows.at[pl.ds(j * CH, CH)], sem))
    for c in copies:
        c.wait()

    wv = wbv[...]
    w = wv[0]
    bb = wv[1]
    lanes = lax.iota(jnp.int32, 16)

    def group_body(g, carry):
        acc = jnp.zeros((16,), jnp.float32)
        for r in range(16):
            i = g * 16 + r
            a0 = srows[i, pl.ds(0, 16)] * trows[i, pl.ds(0, 16)]
            a1 = srows[i, pl.ds(16, 16)] * trows[i, pl.ds(16, 16)]
            a2 = srows[i, pl.ds(32, 16)] * trows[i, pl.ds(32, 16)]
            a3 = srows[i, pl.ds(48, 16)] * trows[i, pl.ds(48, 16)]
            s = jnp.sum((a0 + a1) + (a2 + a3))
            acc = jnp.where(lanes == r, s, acc)
        outv[pl.ds(g * 16, 16)] = acc * w + bb
        return carry

    lax.fori_loop(0, BPW // 16, group_body, 0)

    pltpu.sync_copy(outv, out_hbm.at[pl.ds(base, BPW)])


@jax.jit
def _sc_call(src3, tgt3, table, wb):
    f = pl.kernel(
        _sc_body,
        mesh=plsc.VectorSubcoreMesh(core_axis_name="c", subcore_axis_name="s"),
        out_type=jax.ShapeDtypeStruct((B,), jnp.float32),
        scratch_types=[
            pltpu.VMEM((BPW,), jnp.int32),
            pltpu.VMEM((BPW,), jnp.int32),
            pltpu.VMEM((BPW, D), jnp.float32),
            pltpu.VMEM((BPW, D), jnp.float32),
            pltpu.VMEM((BPW,), jnp.float32),
            pltpu.VMEM((16,), jnp.float32),
            pltpu.SemaphoreType.DMA,
        ],
        compiler_params=pltpu.CompilerParams(
            needs_layout_passes=False, use_tc_tiling_on_sc=False),
    )
    return f(src3, tgt3, table, wb)


def kernel(sources, targets, table, W, b):
    wb = jnp.zeros((16,), jnp.float32)
    wb = wb.at[0].set(W.reshape(())).at[1].set(b.reshape(()))
    out = _sc_call(sources, targets, table, wb)
    return out.reshape(B, 1)


# speedup vs baseline: 2.0519x; 2.0519x over previous
import jax
import jax.numpy as jnp
from jax import lax
from jax.experimental import pallas as pl
from jax.experimental.pallas import tpu as pltpu
from jax.experimental.pallas import tpu_sc as plsc

B = 16384
D = 64
NC = 2
NS = 16
NW = NC * NS
BPW = B // NW
G = 16
NG = BPW // G
NT = 125000


def _sc_body(src_hbm, tgt_hbm, table3_hbm, wb_hbm, out_hbm,
             sidx, tidx, sbuf, tbuf, outv, wbv, sem):
    cid = lax.axis_index("c")
    sid = lax.axis_index("s")
    wid = sid * NC + cid
    base = wid * BPW

    pltpu.sync_copy(src_hbm.at[pl.ds(base, BPW)], sidx)
    pltpu.sync_copy(tgt_hbm.at[pl.ds(base, BPW)], tidx)
    pltpu.sync_copy(wb_hbm, wbv)

    wv = wbv[...]
    w = wv[0]
    bb = wv[1]
    lanes = lax.iota(jnp.int32, 16)

    def group_body(g, carry):
        sv = sidx[pl.ds(g * G, 16)]
        tv = tidx[pl.ds(g * G, 16)]
        stiles = lax.shift_right_logical(sv, 3)
        ttiles = lax.shift_right_logical(tv, 3)
        subs_v = jnp.bitwise_and(sv, 7)
        subt_v = jnp.bitwise_and(tv, 7)

        for r in range(G):
            pltpu.async_copy(
                table3_hbm.at[stiles[r]], sbuf.at[r], sem)
            pltpu.async_copy(
                table3_hbm.at[ttiles[r]], tbuf.at[r], sem)
        pltpu.make_async_copy(table3_hbm.at[pl.ds(0, G)], sbuf, sem).wait()
        pltpu.make_async_copy(table3_hbm.at[pl.ds(0, G)], tbuf, sem).wait()

        acc = jnp.zeros((16,), jnp.float32)
        for r in range(G):
            sub_s = subs_v[r]
            sub_t = subt_v[r]
            a0 = sbuf[r, sub_s, pl.ds(0, 16)] * tbuf[r, sub_t, pl.ds(0, 16)]
            a1 = sbuf[r, sub_s, pl.ds(16, 16)] * tbuf[r, sub_t, pl.ds(16, 16)]
            a2 = sbuf[r, sub_s, pl.ds(32, 16)] * tbuf[r, sub_t, pl.ds(32, 16)]
            a3 = sbuf[r, sub_s, pl.ds(48, 16)] * tbuf[r, sub_t, pl.ds(48, 16)]
            s = jnp.sum((a0 + a1) + (a2 + a3))
            acc = jnp.where(lanes == r, s, acc)
        outv[pl.ds(g * G, 16)] = acc * w + bb
        return carry

    lax.fori_loop(0, NG, group_body, 0)

    pltpu.sync_copy(outv, out_hbm.at[pl.ds(base, BPW)])


@jax.jit
def _sc_call(sources, targets, table3, wb):
    f = pl.kernel(
        _sc_body,
        mesh=plsc.VectorSubcoreMesh(core_axis_name="c", subcore_axis_name="s"),
        out_type=jax.ShapeDtypeStruct((B,), jnp.float32),
        scratch_types=[
            pltpu.VMEM((BPW,), jnp.int32),
            pltpu.VMEM((BPW,), jnp.int32),
            pltpu.VMEM((G, 8, D), jnp.float32),
            pltpu.VMEM((G, 8, D), jnp.float32),
            pltpu.VMEM((BPW,), jnp.float32),
            pltpu.VMEM((16,), jnp.float32),
            pltpu.SemaphoreType.DMA,
        ],
        compiler_params=pltpu.CompilerParams(
            needs_layout_passes=False, use_tc_tiling_on_sc=True),
    )
    return f(sources, targets, table3, wb)


def kernel(sources, targets, table, W, b):
    wb = jnp.zeros((16,), jnp.float32)
    wb = wb.at[0].set(W.reshape(())).at[1].set(b.reshape(()))
    out = _sc_call(sources, targets, table.reshape(NT, 8, D), wb)
    return out.reshape(B, 1)


# speedup vs baseline: 2.1385x; 1.0422x over previous
import jax
import jax.numpy as jnp
from jax import lax
from jax.experimental import pallas as pl
from jax.experimental.pallas import tpu as pltpu
from jax.experimental.pallas import tpu_sc as plsc

B = 16384
D = 64
NC = 2
NS = 16
NW = NC * NS
BPW = B // NW
G = 16
NG = BPW // G
NT = 125000


def _sc_body(src_hbm, tgt_hbm, table3_hbm, wb_hbm, out_hbm,
             sidx, tidx, sbuf, tbuf, outv, wbv, sem):
    cid = lax.axis_index("c")
    sid = lax.axis_index("s")
    wid = sid * NC + cid
    base = wid * BPW

    pltpu.sync_copy(src_hbm.at[pl.ds(base, BPW)], sidx)
    pltpu.sync_copy(tgt_hbm.at[pl.ds(base, BPW)], tidx)
    pltpu.sync_copy(wb_hbm, wbv)

    wv = wbv[...]
    w = wv[0]
    bb = wv[1]
    lanes = lax.iota(jnp.int32, 16)

    def fire(g, which):
        stiles = lax.shift_right_logical(sidx[pl.ds(g * G, 16)], 3)
        ttiles = lax.shift_right_logical(tidx[pl.ds(g * G, 16)], 3)
        for r in range(G):
            pltpu.async_copy(
                table3_hbm.at[stiles[r]], sbuf.at[which, r], sem.at[which])
            pltpu.async_copy(
                table3_hbm.at[ttiles[r]], tbuf.at[which, r], sem.at[which])

    fire(0, 0)

    def group_body(g, carry):
        which = lax.rem(g, 2)

        @pl.when(g + 1 < NG)
        def _():
            fire(g + 1, lax.rem(g + 1, 2))

        pltpu.make_async_copy(
            table3_hbm.at[pl.ds(0, G)], sbuf.at[0], sem.at[which]).wait()
        pltpu.make_async_copy(
            table3_hbm.at[pl.ds(0, G)], tbuf.at[0], sem.at[which]).wait()

        subs_v = jnp.bitwise_and(sidx[pl.ds(g * G, 16)], 7)
        subt_v = jnp.bitwise_and(tidx[pl.ds(g * G, 16)], 7)
        acc = jnp.zeros((16,), jnp.float32)
        for r in range(G):
            sub_s = subs_v[r]
            sub_t = subt_v[r]
            a0 = sbuf[which, r, sub_s, pl.ds(0, 16)] * tbuf[which, r, sub_t, pl.ds(0, 16)]
            a1 = sbuf[which, r, sub_s, pl.ds(16, 16)] * tbuf[which, r, sub_t, pl.ds(16, 16)]
            a2 = sbuf[which, r, sub_s, pl.ds(32, 16)] * tbuf[which, r, sub_t, pl.ds(32, 16)]
            a3 = sbuf[which, r, sub_s, pl.ds(48, 16)] * tbuf[which, r, sub_t, pl.ds(48, 16)]
            s = jnp.sum((a0 + a1) + (a2 + a3))
            acc = jnp.where(lanes == r, s, acc)
        outv[pl.ds(g * G, 16)] = acc * w + bb
        return carry

    lax.fori_loop(0, NG, group_body, 0)

    pltpu.sync_copy(outv, out_hbm.at[pl.ds(base, BPW)])


@jax.jit
def _sc_call(sources, targets, table3, wb):
    f = pl.kernel(
        _sc_body,
        mesh=plsc.VectorSubcoreMesh(core_axis_name="c", subcore_axis_name="s"),
        out_type=jax.ShapeDtypeStruct((B,), jnp.float32),
        scratch_types=[
            pltpu.VMEM((BPW,), jnp.int32),
            pltpu.VMEM((BPW,), jnp.int32),
            pltpu.VMEM((2, G, 8, D), jnp.float32),
            pltpu.VMEM((2, G, 8, D), jnp.float32),
            pltpu.VMEM((BPW,), jnp.float32),
            pltpu.VMEM((16,), jnp.float32),
            pltpu.SemaphoreType.DMA((2,)),
        ],
        compiler_params=pltpu.CompilerParams(
            needs_layout_passes=False, use_tc_tiling_on_sc=True),
    )
    return f(sources, targets, table3, wb)


def kernel(sources, targets, table, W, b):
    wb = jnp.zeros((16,), jnp.float32)
    wb = wb.at[0].set(W.reshape(())).at[1].set(b.reshape(()))
    out = _sc_call(sources, targets, table.reshape(NT, 8, D), wb)
    return out.reshape(B, 1)
